# baseline (speedup 1.0000x reference)
import functools

import numpy as np
import jax
import jax.numpy as jnp
from jax import lax
from jax.experimental import pallas as pl
from jax.experimental.pallas import tpu as pltpu
from jax.experimental.pallas import tpu_sc as plsc

B, N, D = 4, 8192, 768
BLK = 2048
L = 16
NCH = N // L
SHIFT = 21
NBINS = 512
CAND = 512
IDS_PAD = 288


def _mask_ratio_const() -> float:
    rs = np.random.RandomState(42)
    lo, hi, loc, scale = 0.7, 1.0, 1.0, 0.25
    v = rs.normal(loc, scale)
    while not (lo <= v <= hi):
        v = rs.normal(loc, scale)
    return float(v)


NUM_KEEP = int(np.round((1.0 - _mask_ratio_const()) * N))


def _vsort_at(ck, cv, i):
    ks, vs = plsc.sort_key_val(ck[pl.ds(i * L, L)], cv[pl.ds(i * L, L)])
    ck[pl.ds(i * L, L)] = ks
    cv[pl.ds(i * L, L)] = vs


def _cmp_exchange(ck, cv, ia, ib):
    ka = ck[pl.ds(ia * L, L)]
    va = cv[pl.ds(ia * L, L)]
    kb = ck[pl.ds(ib * L, L)]
    vb = cv[pl.ds(ib * L, L)]
    sel = ka <= kb
    ck[pl.ds(ia * L, L)] = jnp.minimum(ka, kb)
    ck[pl.ds(ib * L, L)] = jnp.maximum(ka, kb)
    cv[pl.ds(ia * L, L)] = jnp.where(sel, va, vb)
    cv[pl.ds(ib * L, L)] = jnp.where(sel, vb, va)


def _select_row(u_v, hist_v, ck_v, cv_v):
    lanes = lax.iota(jnp.int32, L)
    ones = jnp.ones((L,), jnp.int32)
    lane_base = lanes * NBINS

    def z(i, _):
        hist_v[pl.ds(i * L, L)] = jnp.zeros((L,), jnp.int32)
        return 0
    lax.fori_loop(0, NBINS, z, 0)

    def h(k, _):
        bits = plsc.bitcast(u_v[pl.ds(k * L, L)], jnp.int32)
        bn = lax.shift_right_logical(bits, SHIFT)
        plsc.addupdate_scatter(hist_v, [lane_base + bn], ones)
        return 0
    lax.fori_loop(0, NCH, h, 0)

    def t(j, carry):
        run, nb = carry
        def tl(l, a):
            return a + hist_v[pl.ds(l * NBINS + j * L, L)]
        acc = lax.fori_loop(0, L, tl, jnp.zeros((L,), jnp.int32))
        cum = plsc.cumsum(acc) + run
        nb = nb + jnp.sum(jnp.where(cum < NUM_KEEP, 1, 0).astype(jnp.int32))
        return cum[L - 1], nb
    _, bstar = lax.fori_loop(0, NBINS // L, t, (jnp.int32(0), jnp.int32(0)))

    inf16 = jnp.full((L,), jnp.inf, jnp.float32)
    def pf(i, _):
        ck_v[pl.ds(i * L, L)] = inf16
        cv_v[pl.ds(i * L, L)] = jnp.zeros((L,), jnp.int32)
        return 0
    lax.fori_loop(0, CAND // L, pf, 0)

    def cp(k, wp):
        v = u_v[pl.ds(k * L, L)]
        bits = plsc.bitcast(v, jnp.int32)
        bn = lax.shift_right_logical(bits, SHIFT)
        pred = bn <= bstar
        pi = jnp.where(pred, 1, 0).astype(jnp.int32)
        pos = wp + plsc.cumsum(pi) - 1
        plsc.store_scatter(ck_v, [pos], v, mask=pred)
        plsc.store_scatter(cv_v, [pos], lanes + k * L, mask=pred)
        return wp + plsc.all_reduce_population_count(pred)[0]
    lax.fori_loop(0, NCH, cp, jnp.int32(0))

    nv = CAND // L

    def s0(i, _):
        _vsort_at(ck_v, cv_v, i)
        return 0
    lax.fori_loop(0, nv, s0, 0)

    m = 1
    while m < nv:
        npairs = nv // (2 * m)
        half = (m + 1) // 2

        def rev_body(t_, _, m=m, half=half):
            p = t_ // half
            i = t_ % half
            base = p * 2 * m + m
            j1 = base + i
            j2 = base + m - 1 - i
            a = ck_v[pl.ds(j1 * L, L)]
            av = cv_v[pl.ds(j1 * L, L)]
            bk = ck_v[pl.ds(j2 * L, L)]
            bv = cv_v[pl.ds(j2 * L, L)]
            ck_v[pl.ds(j1 * L, L)] = jnp.flip(bk, axis=0)
            cv_v[pl.ds(j1 * L, L)] = jnp.flip(bv, axis=0)
            ck_v[pl.ds(j2 * L, L)] = jnp.flip(a, axis=0)
            cv_v[pl.ds(j2 * L, L)] = jnp.flip(av, axis=0)
            return 0
        lax.fori_loop(0, npairs * half, rev_body, 0)

        j = m
        while j >= 1:
            def ex_body(t_, _, m=m, j=j):
                p = t_ // m
                q = t_ % m
                i = (q // j) * 2 * j + (q % j)
                base = p * 2 * m
                _cmp_exchange(ck_v, cv_v, base + i, base + i + j)
                return 0
            lax.fori_loop(0, npairs * m, ex_body, 0)
            j //= 2

        def sl(i, _):
            _vsort_at(ck_v, cv_v, i)
            return 0
        lax.fori_loop(0, nv, sl, 0)
        m *= 2


NW = 32
SPW = 40
SPB = (NW // B) * SPW


def _sel_body(u_hbm, ids_hbm, ids2_hbm, u_v, hist_v, ck_v, cv_v, ids_v, ids2_v):
    c = lax.axis_index("c")
    s = lax.axis_index("s")
    b = 2 * c + s

    @pl.when(s < 2)
    def _():
        pltpu.sync_copy(u_hbm.at[b], u_v)
        _select_row(u_v, hist_v, ck_v, cv_v)

        for i in range(IDS_PAD // L):
            ids_v[pl.ds(i * L, L)] = cv_v[pl.ds(i * L, L)]

        base = b * N
        lanes = lax.iota(jnp.int32, L)
        _lv = cv_v[pl.ds(((NUM_KEEP - 1) // L) * L, L)]
        last = _lv[(NUM_KEEP - 1) % L] + base
        for i in range(IDS_PAD // L):
            v = cv_v[pl.ds(i * L, L)] + base
            nvalid = NUM_KEEP - i * L
            if nvalid >= L:
                ids2_v[pl.ds(i * L, L)] = v
            else:
                ids2_v[pl.ds(i * L, L)] = jnp.where(lanes < nvalid, v, last)
        lastv = jnp.zeros((L,), jnp.int32) + last
        for i in range(IDS_PAD // L, SPB // L):
            ids2_v[pl.ds(i * L, L)] = lastv

        pltpu.sync_copy(ids_v, ids_hbm.at[b])
        pltpu.sync_copy(ids2_v, ids2_hbm.at[pl.ds(b * SPB, SPB)])


@functools.cache
def _make_selection():
    mesh = plsc.VectorSubcoreMesh(core_axis_name="c", subcore_axis_name="s",
                                  num_cores=2, num_subcores=16)
    return pl.kernel(
        _sel_body,
        out_type=(
            jax.ShapeDtypeStruct((B, IDS_PAD), jnp.int32),
            jax.ShapeDtypeStruct((B * SPB,), jnp.int32),
        ),
        mesh=mesh,
        scratch_types=(
            pltpu.VMEM((N,), jnp.float32),
            pltpu.VMEM((NBINS * L,), jnp.int32),
            pltpu.VMEM((CAND,), jnp.float32),
            pltpu.VMEM((CAND,), jnp.int32),
            pltpu.VMEM((IDS_PAD,), jnp.int32),
            pltpu.VMEM((SPB,), jnp.int32),
        ),
        compiler_params=pltpu.CompilerParams(needs_layout_passes=False),
    )


SCHK = 8
SNC = SPW // SCHK


def _scat_body(ids2_hbm, x_hbm, out_ref, idx_v, rows_v, gsem, ssem):
    c = lax.axis_index("c")
    s = lax.axis_index("s")
    w = c * 16 + s
    pltpu.sync_copy(ids2_hbm.at[w], idx_v)
    gathers = [
        pltpu.async_copy(x_hbm.at[idx_v.at[k]], rows_v.at[k], gsem)
        for k in range(SNC)
    ]
    scatters = []
    for k in range(SNC):
        gathers[k].wait()
        scatters.append(
            pltpu.async_copy(rows_v.at[k], out_ref.at[idx_v.at[k]], ssem))
    for cp in scatters:
        cp.wait()


@functools.cache
def _make_scatter():
    mesh = plsc.VectorSubcoreMesh(core_axis_name="c", subcore_axis_name="s",
                                  num_cores=2, num_subcores=16)
    return pl.kernel(
        _scat_body,
        out_type=(),
        mesh=mesh,
        scratch_types=(
            pltpu.VMEM((SNC, SCHK), jnp.int32),
            pltpu.VMEM((SNC, SCHK, D), jnp.float32),
            pltpu.SemaphoreType.DMA,
            pltpu.SemaphoreType.DMA,
        ),
        compiler_params=pltpu.CompilerParams(needs_layout_passes=False),
    )


def _fill_body(tok_ref, o_ref):
    tok = tok_ref[...]
    o_ref[...] = jnp.broadcast_to(tok[None], o_ref.shape)


def _fill(mask_token):
    return pl.pallas_call(
        _fill_body,
        grid=(B, N // BLK),
        in_specs=[pl.BlockSpec((1, D), lambda b, j: (0, 0))],
        out_specs=pl.BlockSpec((1, BLK, D), lambda b, j: (b, j, 0)),
        out_shape=jax.ShapeDtypeStruct((B, N, D), jnp.float32),
    )(mask_token)


def kernel(x, mask_token):
    u = jax.random.uniform(jax.random.key(123), (B, N, 1), dtype=x.dtype)
    ids_pad, ids2 = _make_selection()(u[:, :, 0])
    prev_ids = ids_pad[:, :NUM_KEEP, None]
    filled = _fill(mask_token)
    out_ref = jax.new_ref(filled.reshape(B * N, D))
    _make_scatter()(ids2.reshape(NW, SNC, SCHK), x.reshape(B * N, D), out_ref)
    out = out_ref[...].reshape(B, N, D)
    return (out, prev_ids)


# speedup vs baseline: 1.0174x; 1.0174x over previous
import functools

import numpy as np
import jax
import jax.numpy as jnp
from jax import lax
from jax.experimental import pallas as pl
from jax.experimental.pallas import tpu as pltpu
from jax.experimental.pallas import tpu_sc as plsc

B, N, D = 4, 8192, 768
BLK = 1024
L = 16
NCH = N // L
SHIFT = 21
NBINS = 512
CAND = 512
IDS_PAD = 288


def _mask_ratio_const() -> float:
    rs = np.random.RandomState(42)
    lo, hi, loc, scale = 0.7, 1.0, 1.0, 0.25
    v = rs.normal(loc, scale)
    while not (lo <= v <= hi):
        v = rs.normal(loc, scale)
    return float(v)


NUM_KEEP = int(np.round((1.0 - _mask_ratio_const()) * N))


def _vsort_at(ck, cv, i):
    ks, vs = plsc.sort_key_val(ck[pl.ds(i * L, L)], cv[pl.ds(i * L, L)])
    ck[pl.ds(i * L, L)] = ks
    cv[pl.ds(i * L, L)] = vs


def _cmp_exchange(ck, cv, ia, ib):
    ka = ck[pl.ds(ia * L, L)]
    va = cv[pl.ds(ia * L, L)]
    kb = ck[pl.ds(ib * L, L)]
    vb = cv[pl.ds(ib * L, L)]
    sel = ka <= kb
    ck[pl.ds(ia * L, L)] = jnp.minimum(ka, kb)
    ck[pl.ds(ib * L, L)] = jnp.maximum(ka, kb)
    cv[pl.ds(ia * L, L)] = jnp.where(sel, va, vb)
    cv[pl.ds(ib * L, L)] = jnp.where(sel, vb, va)


def _select_row(u_v, hist_v, ck_v, cv_v):
    lanes = lax.iota(jnp.int32, L)
    ones = jnp.ones((L,), jnp.int32)
    lane_base = lanes * NBINS

    def z(i, _):
        hist_v[pl.ds(i * L, L)] = jnp.zeros((L,), jnp.int32)
        return 0
    lax.fori_loop(0, NBINS, z, 0)

    def h(k, _):
        bits = plsc.bitcast(u_v[pl.ds(k * L, L)], jnp.int32)
        bn = lax.shift_right_logical(bits, SHIFT)
        plsc.addupdate_scatter(hist_v, [lane_base + bn], ones)
        return 0
    lax.fori_loop(0, NCH, h, 0)

    def t(j, carry):
        run, nb = carry
        def tl(l, a):
            return a + hist_v[pl.ds(l * NBINS + j * L, L)]
        acc = lax.fori_loop(0, L, tl, jnp.zeros((L,), jnp.int32))
        cum = plsc.cumsum(acc) + run
        nb = nb + jnp.sum(jnp.where(cum < NUM_KEEP, 1, 0).astype(jnp.int32))
        return cum[L - 1], nb
    _, bstar = lax.fori_loop(0, NBINS // L, t, (jnp.int32(0), jnp.int32(0)))

    inf16 = jnp.full((L,), jnp.inf, jnp.float32)
    def pf(i, _):
        ck_v[pl.ds(i * L, L)] = inf16
        cv_v[pl.ds(i * L, L)] = jnp.zeros((L,), jnp.int32)
        return 0
    lax.fori_loop(0, CAND // L, pf, 0)

    def cp(k, wp):
        v = u_v[pl.ds(k * L, L)]
        bits = plsc.bitcast(v, jnp.int32)
        bn = lax.shift_right_logical(bits, SHIFT)
        pred = bn <= bstar
        pi = jnp.where(pred, 1, 0).astype(jnp.int32)
        pos = wp + plsc.cumsum(pi) - 1
        plsc.store_scatter(ck_v, [pos], v, mask=pred)
        plsc.store_scatter(cv_v, [pos], lanes + k * L, mask=pred)
        return wp + plsc.all_reduce_population_count(pred)[0]
    lax.fori_loop(0, NCH, cp, jnp.int32(0))

    nv = CAND // L

    def s0(i, _):
        _vsort_at(ck_v, cv_v, i)
        return 0
    lax.fori_loop(0, nv, s0, 0)

    m = 1
    while m < nv:
        npairs = nv // (2 * m)
        half = (m + 1) // 2

        def rev_body(t_, _, m=m, half=half):
            p = t_ // half
            i = t_ % half
            base = p * 2 * m + m
            j1 = base + i
            j2 = base + m - 1 - i
            a = ck_v[pl.ds(j1 * L, L)]
            av = cv_v[pl.ds(j1 * L, L)]
            bk = ck_v[pl.ds(j2 * L, L)]
            bv = cv_v[pl.ds(j2 * L, L)]
            ck_v[pl.ds(j1 * L, L)] = jnp.flip(bk, axis=0)
            cv_v[pl.ds(j1 * L, L)] = jnp.flip(bv, axis=0)
            ck_v[pl.ds(j2 * L, L)] = jnp.flip(a, axis=0)
            cv_v[pl.ds(j2 * L, L)] = jnp.flip(av, axis=0)
            return 0
        lax.fori_loop(0, npairs * half, rev_body, 0)

        j = m
        while j >= 1:
            def ex_body(t_, _, m=m, j=j):
                p = t_ // m
                q = t_ % m
                i = (q // j) * 2 * j + (q % j)
                base = p * 2 * m
                _cmp_exchange(ck_v, cv_v, base + i, base + i + j)
                return 0
            lax.fori_loop(0, npairs * m, ex_body, 0)
            j //= 2

        def sl(i, _):
            _vsort_at(ck_v, cv_v, i)
            return 0
        lax.fori_loop(0, nv, sl, 0)
        m *= 2


NW = 32
SPW = 40
SPB = (NW // B) * SPW


def _sel_body(u_hbm, ids_hbm, ids2_hbm, u_v, hist_v, ck_v, cv_v, ids_v, ids2_v):
    c = lax.axis_index("c")
    s = lax.axis_index("s")
    b = 2 * c + s

    @pl.when(s < 2)
    def _():
        pltpu.sync_copy(u_hbm.at[b], u_v)
        _select_row(u_v, hist_v, ck_v, cv_v)

        for i in range(IDS_PAD // L):
            ids_v[pl.ds(i * L, L)] = cv_v[pl.ds(i * L, L)]

        base = b * N
        lanes = lax.iota(jnp.int32, L)
        _lv = cv_v[pl.ds(((NUM_KEEP - 1) // L) * L, L)]
        last = _lv[(NUM_KEEP - 1) % L] + base
        for i in range(IDS_PAD // L):
            v = cv_v[pl.ds(i * L, L)] + base
            nvalid = NUM_KEEP - i * L
            if nvalid >= L:
                ids2_v[pl.ds(i * L, L)] = v
            else:
                ids2_v[pl.ds(i * L, L)] = jnp.where(lanes < nvalid, v, last)
        lastv = jnp.zeros((L,), jnp.int32) + last
        for i in range(IDS_PAD // L, SPB // L):
            ids2_v[pl.ds(i * L, L)] = lastv

        pltpu.sync_copy(ids_v, ids_hbm.at[b])
        pltpu.sync_copy(ids2_v, ids2_hbm.at[pl.ds(b * SPB, SPB)])


@functools.cache
def _make_selection():
    mesh = plsc.VectorSubcoreMesh(core_axis_name="c", subcore_axis_name="s",
                                  num_cores=2, num_subcores=16)
    return pl.kernel(
        _sel_body,
        out_type=(
            jax.ShapeDtypeStruct((B, IDS_PAD), jnp.int32),
            jax.ShapeDtypeStruct((B * SPB,), jnp.int32),
        ),
        mesh=mesh,
        scratch_types=(
            pltpu.VMEM((N,), jnp.float32),
            pltpu.VMEM((NBINS * L,), jnp.int32),
            pltpu.VMEM((CAND,), jnp.float32),
            pltpu.VMEM((CAND,), jnp.int32),
            pltpu.VMEM((IDS_PAD,), jnp.int32),
            pltpu.VMEM((SPB,), jnp.int32),
        ),
        compiler_params=pltpu.CompilerParams(needs_layout_passes=False),
    )


SCHK = 8
SNC = SPW // SCHK


def _scat_body(ids2_hbm, x_hbm, out_ref, idx_v, rows_v, gsem, ssem):
    c = lax.axis_index("c")
    s = lax.axis_index("s")
    w = c * 16 + s
    pltpu.sync_copy(ids2_hbm.at[w], idx_v)
    gathers = [
        pltpu.async_copy(x_hbm.at[idx_v.at[k]], rows_v.at[k], gsem)
        for k in range(SNC)
    ]
    scatters = []
    for k in range(SNC):
        gathers[k].wait()
        scatters.append(
            pltpu.async_copy(rows_v.at[k], out_ref.at[idx_v.at[k]], ssem))
    for cp in scatters:
        cp.wait()


@functools.cache
def _make_scatter():
    mesh = plsc.VectorSubcoreMesh(core_axis_name="c", subcore_axis_name="s",
                                  num_cores=2, num_subcores=16)
    return pl.kernel(
        _scat_body,
        out_type=(),
        mesh=mesh,
        scratch_types=(
            pltpu.VMEM((SNC, SCHK), jnp.int32),
            pltpu.VMEM((SNC, SCHK, D), jnp.float32),
            pltpu.SemaphoreType.DMA,
            pltpu.SemaphoreType.DMA,
        ),
        compiler_params=pltpu.CompilerParams(needs_layout_passes=False),
    )


def _fill_body(tok_ref, o_ref):
    tok = tok_ref[...]
    o_ref[...] = jnp.broadcast_to(tok[None], o_ref.shape)


def _fill(mask_token):
    return pl.pallas_call(
        _fill_body,
        grid=(B, N // BLK),
        in_specs=[pl.BlockSpec((1, D), lambda b, j: (0, 0))],
        out_specs=pl.BlockSpec((1, BLK, D), lambda b, j: (b, j, 0)),
        out_shape=jax.ShapeDtypeStruct((B, N, D), jnp.float32),
    )(mask_token)


def kernel(x, mask_token):
    u = jax.random.uniform(jax.random.key(123), (B, N, 1), dtype=x.dtype)
    ids_pad, ids2 = _make_selection()(u[:, :, 0])
    prev_ids = ids_pad[:, :NUM_KEEP, None]
    filled = _fill(mask_token)
    out_ref = jax.new_ref(filled.reshape(B * N, D))
    _make_scatter()(ids2.reshape(NW, SNC, SCHK), x.reshape(B * N, D), out_ref)
    out = out_ref[...].reshape(B, N, D)
    return (out, prev_ids)


# speedup vs baseline: 1.0408x; 1.0230x over previous
import functools

import numpy as np
import jax
import jax.numpy as jnp
from jax import lax
from jax.experimental import pallas as pl
from jax.experimental.pallas import tpu as pltpu
from jax.experimental.pallas import tpu_sc as plsc

B, N, D = 4, 8192, 768
BLK = 1024
L = 16
NCH = N // L
SHIFT = 21
NBINS = 512
CAND = 512
IDS_PAD = 288


def _mask_ratio_const() -> float:
    rs = np.random.RandomState(42)
    lo, hi, loc, scale = 0.7, 1.0, 1.0, 0.25
    v = rs.normal(loc, scale)
    while not (lo <= v <= hi):
        v = rs.normal(loc, scale)
    return float(v)


NUM_KEEP = int(np.round((1.0 - _mask_ratio_const()) * N))


def _vsort_at(ck, cv, i):
    ks, vs = plsc.sort_key_val(ck[pl.ds(i * L, L)], cv[pl.ds(i * L, L)])
    ck[pl.ds(i * L, L)] = ks
    cv[pl.ds(i * L, L)] = vs


def _cmp_exchange(ck, cv, ia, ib):
    ka = ck[pl.ds(ia * L, L)]
    va = cv[pl.ds(ia * L, L)]
    kb = ck[pl.ds(ib * L, L)]
    vb = cv[pl.ds(ib * L, L)]
    sel = ka <= kb
    ck[pl.ds(ia * L, L)] = jnp.minimum(ka, kb)
    ck[pl.ds(ib * L, L)] = jnp.maximum(ka, kb)
    cv[pl.ds(ia * L, L)] = jnp.where(sel, va, vb)
    cv[pl.ds(ib * L, L)] = jnp.where(sel, vb, va)


def _select_row(u_v, hist_v, ck_v, cv_v):
    lanes = lax.iota(jnp.int32, L)
    ones = jnp.ones((L,), jnp.int32)
    lane_base = lanes * NBINS

    def z(i, _):
        hist_v[pl.ds(i * L, L)] = jnp.zeros((L,), jnp.int32)
        return 0
    lax.fori_loop(0, NBINS, z, 0)

    def h(k, _):
        bits = plsc.bitcast(u_v[pl.ds(k * L, L)], jnp.int32)
        bn = lax.shift_right_logical(bits, SHIFT)
        plsc.addupdate_scatter(hist_v, [lane_base + bn], ones)
        return 0
    lax.fori_loop(0, NCH, h, 0)

    def t(j, carry):
        run, nb = carry
        def tl(l, a):
            return a + hist_v[pl.ds(l * NBINS + j * L, L)]
        acc = lax.fori_loop(0, L, tl, jnp.zeros((L,), jnp.int32))
        cum = plsc.cumsum(acc) + run
        nb = nb + jnp.sum(jnp.where(cum < NUM_KEEP, 1, 0).astype(jnp.int32))
        return cum[L - 1], nb
    _, bstar = lax.fori_loop(0, NBINS // L, t, (jnp.int32(0), jnp.int32(0)))

    inf16 = jnp.full((L,), jnp.inf, jnp.float32)
    def pf(i, _):
        ck_v[pl.ds(i * L, L)] = inf16
        cv_v[pl.ds(i * L, L)] = jnp.zeros((L,), jnp.int32)
        return 0
    lax.fori_loop(0, CAND // L, pf, 0)

    def cp(k, wp):
        v = u_v[pl.ds(k * L, L)]
        bits = plsc.bitcast(v, jnp.int32)
        bn = lax.shift_right_logical(bits, SHIFT)
        pred = bn <= bstar
        pi = jnp.where(pred, 1, 0).astype(jnp.int32)
        pos = wp + plsc.cumsum(pi) - 1
        plsc.store_scatter(ck_v, [pos], v, mask=pred)
        plsc.store_scatter(cv_v, [pos], lanes + k * L, mask=pred)
        return wp + plsc.all_reduce_population_count(pred)[0]
    lax.fori_loop(0, NCH, cp, jnp.int32(0))

    nv = CAND // L

    def s0(i, _):
        _vsort_at(ck_v, cv_v, i)
        return 0
    lax.fori_loop(0, nv, s0, 0)

    m = 1
    while m < nv:
        npairs = nv // (2 * m)
        half = (m + 1) // 2

        def rev_body(t_, _, m=m, half=half):
            p = t_ // half
            i = t_ % half
            base = p * 2 * m + m
            j1 = base + i
            j2 = base + m - 1 - i
            a = ck_v[pl.ds(j1 * L, L)]
            av = cv_v[pl.ds(j1 * L, L)]
            bk = ck_v[pl.ds(j2 * L, L)]
            bv = cv_v[pl.ds(j2 * L, L)]
            ck_v[pl.ds(j1 * L, L)] = jnp.flip(bk, axis=0)
            cv_v[pl.ds(j1 * L, L)] = jnp.flip(bv, axis=0)
            ck_v[pl.ds(j2 * L, L)] = jnp.flip(a, axis=0)
            cv_v[pl.ds(j2 * L, L)] = jnp.flip(av, axis=0)
            return 0
        lax.fori_loop(0, npairs * half, rev_body, 0)

        j = m
        while j >= 1:
            def ex_body(t_, _, m=m, j=j):
                p = t_ // m
                q = t_ % m
                i = (q // j) * 2 * j + (q % j)
                base = p * 2 * m
                _cmp_exchange(ck_v, cv_v, base + i, base + i + j)
                return 0
            lax.fori_loop(0, npairs * m, ex_body, 0)
            j //= 2

        def sl(i, _):
            _vsort_at(ck_v, cv_v, i)
            return 0
        lax.fori_loop(0, nv, sl, 0)
        m *= 2


NW = 32
SPW = 40
SPB = (NW // B) * SPW


def _sel_body(u_hbm, ids_hbm, ids2_hbm, u_v, hist_v, ck_v, cv_v, ids_v, ids2_v):
    c = lax.axis_index("c")
    s = lax.axis_index("s")
    b = 2 * c + s

    @pl.when(s < 2)
    def _():
        pltpu.sync_copy(u_hbm.at[b], u_v)
        _select_row(u_v, hist_v, ck_v, cv_v)

        for i in range(IDS_PAD // L):
            ids_v[pl.ds(i * L, L)] = cv_v[pl.ds(i * L, L)]

        base = b * N
        lanes = lax.iota(jnp.int32, L)
        _lv = cv_v[pl.ds(((NUM_KEEP - 1) // L) * L, L)]
        last = _lv[(NUM_KEEP - 1) % L] + base
        for i in range(IDS_PAD // L):
            v = cv_v[pl.ds(i * L, L)] + base
            nvalid = NUM_KEEP - i * L
            if nvalid >= L:
                ids2_v[pl.ds(i * L, L)] = v
            else:
                ids2_v[pl.ds(i * L, L)] = jnp.where(lanes < nvalid, v, last)
        lastv = jnp.zeros((L,), jnp.int32) + last
        for i in range(IDS_PAD // L, SPB // L):
            ids2_v[pl.ds(i * L, L)] = lastv

        pltpu.sync_copy(ids_v, ids_hbm.at[b])
        pltpu.sync_copy(ids2_v, ids2_hbm.at[pl.ds(b * SPB, SPB)])


@functools.cache
def _make_selection():
    mesh = plsc.VectorSubcoreMesh(core_axis_name="c", subcore_axis_name="s",
                                  num_cores=2, num_subcores=16)
    return pl.kernel(
        _sel_body,
        out_type=(
            jax.ShapeDtypeStruct((B, IDS_PAD), jnp.int32),
            jax.ShapeDtypeStruct((B * SPB,), jnp.int32),
        ),
        mesh=mesh,
        scratch_types=(
            pltpu.VMEM((N,), jnp.float32),
            pltpu.VMEM((NBINS * L,), jnp.int32),
            pltpu.VMEM((CAND,), jnp.float32),
            pltpu.VMEM((CAND,), jnp.int32),
            pltpu.VMEM((IDS_PAD,), jnp.int32),
            pltpu.VMEM((SPB,), jnp.int32),
        ),
        compiler_params=pltpu.CompilerParams(needs_layout_passes=False),
    )


SCHK = 8
SNC = SPW // SCHK


def _scat_body(ids2_hbm, x_hbm, out_ref, idx_v, rows_v, sem):
    c = lax.axis_index("c")
    s = lax.axis_index("s")
    w = c * 16 + s
    pltpu.sync_copy(ids2_hbm.at[pl.ds(w * SPW, SPW)], idx_v)
    pltpu.async_copy(x_hbm.at[idx_v], rows_v, sem).wait()
    pltpu.sync_copy(rows_v, out_ref.at[idx_v])


@functools.cache
def _make_scatter():
    mesh = plsc.VectorSubcoreMesh(core_axis_name="c", subcore_axis_name="s",
                                  num_cores=2, num_subcores=16)
    return pl.kernel(
        _scat_body,
        out_type=(),
        mesh=mesh,
        scratch_types=(
            pltpu.VMEM((SPW,), jnp.int32),
            pltpu.VMEM((SPW, D), jnp.float32),
            pltpu.SemaphoreType.DMA,
        ),
        compiler_params=pltpu.CompilerParams(needs_layout_passes=False),
    )


def _fill_body(tok_ref, o_ref):
    tok = tok_ref[...]
    o_ref[...] = jnp.broadcast_to(tok[None], o_ref.shape)


def _fill(mask_token):
    return pl.pallas_call(
        _fill_body,
        grid=(B, N // BLK),
        in_specs=[pl.BlockSpec((1, D), lambda b, j: (0, 0))],
        out_specs=pl.BlockSpec((1, BLK, D), lambda b, j: (b, j, 0)),
        out_shape=jax.ShapeDtypeStruct((B, N, D), jnp.float32),
    )(mask_token)


def kernel(x, mask_token):
    u = jax.random.uniform(jax.random.key(123), (B, N, 1), dtype=x.dtype)
    ids_pad, ids2 = _make_selection()(u[:, :, 0])
    prev_ids = ids_pad[:, :NUM_KEEP, None]
    filled = _fill(mask_token)
    out_ref = jax.new_ref(filled.reshape(B * N, D))
    _make_scatter()(ids2, x.reshape(B * N, D), out_ref)
    out = out_ref[...].reshape(B, N, D)
    return (out, prev_ids)
